# baseline (speedup 1.0000x reference)
import functools

import jax
import jax.numpy as jnp
import numpy as np
from jax import lax
from jax.experimental import pallas as pl
from jax.experimental.pallas import tpu as pltpu
from jax.experimental.pallas import tpu_sc as plsc

_B, _N, _F = 4, 10000, 128
_K, _FILT = 3, 4
_KP = _K + 1
_NNZ = 320000
_FF = _F * _FILT

_NC, _NS, _L = 2, 16, 16
_BPC = _B // _NC
_C = 80
_CHB = _NNZ // _C
_NCH = _CHB // _NS
_W = _F // 2
_NG = _F // 32
_NP = 10240
_RPT = _NP // _NS
_RC = 64
_NRC = _RPT // _RC
_PSTART = 9984
_NV = _F // _L
_HI = -65536


def _sc_body(xtab, cols3, rows3, vals3, outp,
             acc, gbuf, gpk, colb, rowb, valb, scat, rbuf, rpk,
             msc0, msc1, msr0, msr1, msv0, msv1, gs0, gs1, ss0, ss1):
    cid = lax.axis_index("c")
    sid = lax.axis_index("s")
    row0 = sid * _RPT
    is_last_tile = row0 + _RPT > _N
    msc, msr, msv = (msc0, msc1), (msr0, msr1), (msv0, msv1)
    gs, ss = (gs0, gs1), (ss0, ss1)

    def _zero_rbuf():
        zv = jnp.zeros((_L,), jnp.float32)

        def _zrow(i, carry):
            for q in range(_NV):
                rbuf[i, pl.ds(q * _L, _L)] = zv
            return carry

        lax.fori_loop(0, _RC, _zrow, 0)

    def _pack_rows(nr):
        def _row(i, carry):
            for q in range(_NG):
                a = lax.bitcast_convert_type(rbuf[i, pl.ds(32 * q, _L)],
                                             jnp.int32)
                bb = lax.bitcast_convert_type(
                    rbuf[i, pl.ds(32 * q + _L, _L)], jnp.int32)
                ra = lax.shift_right_logical(
                    a + 32767 + (lax.shift_right_logical(a, 16) & 1), 16)
                rb = bb + 32767 + (lax.shift_right_logical(bb, 16) & 1)
                rpk[i, pl.ds(q * _L, _L)] = ra | (rb & _HI)
            return carry

        lax.fori_loop(0, nr, _row, 0)

    def _unpack_neg_rows(nr):
        def _row(i, carry):
            for q in range(_NG):
                w = rpk[i, pl.ds(q * _L, _L)]
                lo = lax.bitcast_convert_type(w << 16, jnp.float32)
                hi = lax.bitcast_convert_type(w & _HI, jnp.float32)
                rbuf[i, pl.ds(32 * q, _L)] = -lo
                rbuf[i, pl.ds(32 * q + _L, _L)] = -hi
            return carry

        lax.fori_loop(0, nr, _row, 0)

    def _init_chunk(k, init_off, start, nr):
        @pl.when(k >= 2)
        def _():
            pltpu.sync_copy(outp.at[pl.ds(init_off + start, nr)],
                            rpk.at[pl.ds(0, nr)])
            _unpack_neg_rows(nr)

        pltpu.sync_copy(rbuf.at[pl.ds(0, nr)], acc.at[pl.ds(start, nr)])

    def _wb_chunk(wb_off, start, nr):
        pltpu.sync_copy(acc.at[pl.ds(start, nr)], rbuf.at[pl.ds(0, nr)])
        _pack_rows(nr)
        pltpu.sync_copy(rpk.at[pl.ds(0, nr)],
                        outp.at[pl.ds(wb_off + start, nr)])

    def _stage_chunk(b, start, nr):
        pltpu.sync_copy(xtab.at[pl.ds(b * _N + start, nr)],
                        rpk.at[pl.ds(0, nr)])
        pltpu.sync_copy(rpk.at[pl.ds(0, nr)],
                        outp.at[pl.ds(b * _KP * _N + start, nr)])

    def _meta_start(cb, p):
        pltpu.async_copy(cols3.at[cb], colb.at[p], msc[p])
        pltpu.async_copy(rows3.at[cb], rowb.at[p], msr[p])
        pltpu.async_copy(vals3.at[cb], valb.at[p], msv[p])

    def _meta_wait(cb, p):
        pltpu.make_async_copy(cols3.at[cb], colb.at[p], msc[p]).wait()
        pltpu.make_async_copy(rows3.at[cb], rowb.at[p], msr[p]).wait()
        pltpu.make_async_copy(vals3.at[cb], valb.at[p], msv[p]).wait()

    def _adjust(p, offv):
        for q in range(_C // _L):
            sl = pl.ds(q * _L, _L)
            colb[p, 0, sl] = colb[p, 0, sl] + offv

    def _gather_start(p):
        pltpu.async_copy(outp.at[colb.at[p, 0]], gpk.at[p], gs[p])

    def _gather_wait(p):
        pltpu.make_async_copy(outp.at[colb.at[p, 0]], gpk.at[p], gs[p]).wait()

    def _rowcopy(p):
        for q in range(_C // _L):
            sl = pl.ds(q * _L, _L)
            scat[p, 0, sl] = rowb[p, 0, sl]

    def _scale(p, facv):
        def _grp(g, carry):
            ev = valb[p, 0, pl.ds(g * _L, _L)] * facv
            for l in range(_L):
                vv = jnp.full((_L,), ev[l], dtype=jnp.float32)
                e = g * _L + l
                for q in range(_NG):
                    w = gpk[p, e, pl.ds(q * _L, _L)]
                    lo = lax.bitcast_convert_type(w << 16, jnp.float32)
                    hi = lax.bitcast_convert_type(w & _HI, jnp.float32)
                    gbuf[p, e, pl.ds(32 * q, _L)] = lo * vv
                    gbuf[p, e, pl.ds(32 * q + _L, _L)] = hi * vv
            return carry

        lax.fori_loop(0, _C // _L, _grp, 0)

    def _scatter_start(p):
        pltpu.async_copy(gbuf.at[p], acc.at[scat.at[p, 0]], ss[p], add=True)

    def _scatter_wait(p):
        pltpu.make_async_copy(gbuf.at[p], acc.at[scat.at[p, 0]], ss[p]).wait()

    def _edge_phase(b, tab_off, facv):
        cb0 = b * _CHB + sid * _NCH
        offv = jnp.full((_L,), tab_off, dtype=jnp.int32)

        _meta_start(cb0, 0)
        _meta_start(cb0 + 1, 1)
        _meta_wait(cb0, 0)
        _adjust(0, offv)
        _gather_start(0)
        _meta_wait(cb0 + 1, 1)
        _adjust(1, offv)
        _gather_start(1)
        _gather_wait(0)
        _rowcopy(0)
        _scale(0, facv)
        _scatter_start(0)
        _meta_start(cb0 + 2, 0)

        def _pair(i, carry):
            j0 = 2 * i
            for p in range(2):
                j = j0 + p
                o = 1 - p
                _meta_wait(cb0 + j, p)
                _scatter_wait(p)
                _adjust(p, offv)
                _gather_start(p)
                _gather_wait(o)
                _rowcopy(o)
                _scale(o, facv)
                _scatter_start(o)
                _meta_start(cb0 + j + 1, o)
            return carry

        lax.fori_loop(1, _NCH // 2, _pair, 0)

        _meta_wait(cb0 + _NCH, 0)
        _scatter_wait(0)
        _gather_wait(1)
        _rowcopy(1)
        _scale(1, facv)
        _scatter_start(1)
        _scatter_wait(1)

    def _graph(i, carry):
        b = cid * _BPC + i

        def _st(r, c2):
            start = row0 + r * _RC

            @pl.when(start + _RC <= _N)
            def _():
                _stage_chunk(b, start, _RC)

            return c2

        lax.fori_loop(0, _NRC, _st, 0)

        @pl.when(is_last_tile)
        def _():
            _stage_chunk(b, _PSTART, _L)

        def _phase(k, c2):
            tab_off = (b * _KP + (k - 1)) * _N
            init_off = (b * _KP + (k - 2)) * _N
            wb_off = (b * _KP + k) * _N
            facv = (jnp.full((_L,), 1.0, jnp.float32)
                    + lax.convert_element_type(k >= 2, jnp.float32))

            @pl.when(k == 1)
            def _():
                _zero_rbuf()

            def _initr(r, c3):
                start = row0 + r * _RC

                @pl.when(start + _RC <= _N)
                def _():
                    _init_chunk(k, init_off, start, _RC)

                return c3

            lax.fori_loop(0, _NRC, _initr, 0)

            @pl.when(is_last_tile)
            def _():
                _init_chunk(k, init_off, _PSTART, _L)

            plsc.subcore_barrier()

            _edge_phase(b, tab_off, facv)
            plsc.subcore_barrier()

            def _wbr(r, c3):
                start = row0 + r * _RC

                @pl.when(start + _RC <= _N)
                def _():
                    _wb_chunk(wb_off, start, _RC)

                return c3

            lax.fori_loop(0, _NRC, _wbr, 0)

            @pl.when(is_last_tile)
            def _():
                _wb_chunk(wb_off, _PSTART, _L)

            return c2

        lax.fori_loop(1, _K + 1, _phase, 0)
        return carry

    lax.fori_loop(0, _BPC, _graph, 0)


_sc_cheb = functools.partial(
    pl.kernel,
    out_type=jax.ShapeDtypeStruct((_B * _KP * _N, _W), jnp.int32),
    mesh=plsc.VectorSubcoreMesh(
        core_axis_name="c", subcore_axis_name="s",
        num_cores=_NC, num_subcores=_NS),
    compiler_params=pltpu.CompilerParams(use_tc_tiling_on_sc=False),
    scratch_types=[
        pltpu.VMEM_SHARED((_NP, _F), jnp.float32),
        pltpu.VMEM((2, _C, _F), jnp.float32),
        pltpu.VMEM((2, _C, _W), jnp.int32),
        pltpu.VMEM((2, 1, _C), jnp.int32),
        pltpu.VMEM((2, 1, _C), jnp.int32),
        pltpu.VMEM((2, 1, _C), jnp.float32),
        pltpu.VMEM((2, 1, _C), jnp.int32),
        pltpu.VMEM((_RC, _F), jnp.float32),
        pltpu.VMEM((_RC, _W), jnp.int32),
        pltpu.SemaphoreType.DMA,
        pltpu.SemaphoreType.DMA,
        pltpu.SemaphoreType.DMA,
        pltpu.SemaphoreType.DMA,
        pltpu.SemaphoreType.DMA,
        pltpu.SemaphoreType.DMA,
        pltpu.SemaphoreType.DMA,
        pltpu.SemaphoreType.DMA,
        pltpu.SemaphoreType.DMA,
        pltpu.SemaphoreType.DMA,
    ],
)(_sc_body)


_PERM = np.empty(_F, dtype=np.int32)
for _g in range(_NG):
    for _j in range(16):
        _PERM[32 * _g + 2 * _j] = 32 * _g + _j
        _PERM[32 * _g + 2 * _j + 1] = 32 * _g + 16 + _j
_PERML = np.array([32 * (i // 16) + (i % 16) for i in range(_W)],
                  dtype=np.int32)

_BN = 400


def _combine_body(x_ref, x1_ref, x2_ref, x3_ref, w0_ref, wl_ref, wh_ref,
                  b_ref, o_ref):
    a = jnp.dot(x_ref[0], w0_ref[0], preferred_element_type=jnp.float32)
    for k, xr in enumerate((x1_ref, x2_ref, x3_ref)):
        w = xr[0]
        lo = lax.bitcast_convert_type(w << 16, jnp.float32)
        hi = lax.bitcast_convert_type(w & _HI, jnp.float32)
        a = a + jnp.dot(lo, wl_ref[k], preferred_element_type=jnp.float32)
        a = a + jnp.dot(hi, wh_ref[k], preferred_element_type=jnp.float32)
    o_ref[0] = jnp.maximum(a + b_ref[:, :], 0.0)


def _tc_combine(x, xsr, w0, wl, wh, bias2d):
    def _xspec(k):
        return pl.BlockSpec((1, _BN, _W),
                            lambda b, n, k=k: (b * _KP + k, n, 0))

    return pl.pallas_call(
        _combine_body,
        grid=(_B, _N // _BN),
        in_specs=[
            pl.BlockSpec((1, _BN, _F), lambda b, n: (b, n, 0)),
            _xspec(1),
            _xspec(2),
            _xspec(3),
            pl.BlockSpec((1, _F, _FF), lambda b, n: (0, 0, 0)),
            pl.BlockSpec((_K, _W, _FF), lambda b, n: (0, 0, 0)),
            pl.BlockSpec((_K, _W, _FF), lambda b, n: (0, 0, 0)),
            pl.BlockSpec((1, _FF), lambda b, n: (0, 0)),
        ],
        out_specs=pl.BlockSpec((1, _BN, _FF), lambda b, n: (b, n, 0)),
        out_shape=jax.ShapeDtypeStruct((_B, _N, _FF), jnp.float32),
    )(x, xsr, xsr, xsr, w0, wl, wh, bias2d)


def kernel(x, lap_rows, lap_cols, lap_vals, weight, bias):
    xf = x.reshape(_B * _N, _F)
    ybf = xf[:, _PERM].astype(jnp.bfloat16)
    xtab = lax.bitcast_convert_type(ybf.reshape(_B * _N, _W, 2), jnp.int32)
    pad1 = ((0, 1), (0, 0), (0, 0))
    cols3 = jnp.pad(lap_cols.reshape(_B * _CHB, 1, _C).astype(jnp.int32), pad1)
    rows3 = jnp.pad(lap_rows.reshape(_B * _CHB, 1, _C).astype(jnp.int32), pad1)
    vals3 = jnp.pad(lap_vals.reshape(_B * _CHB, 1, _C), pad1)
    xs = _sc_cheb(xtab, cols3, rows3, vals3)

    eye = jnp.eye(_F, dtype=jnp.float32)
    wbig = jnp.stack(
        [(eye[:, :, None] * weight[k][None, None, :]).reshape(_F, _FF)
         for k in range(_K + 1)])
    w0 = wbig[:1]
    wl = wbig[1:, _PERML, :]
    wh = wbig[1:, _PERML + 16, :]
    bias2d = bias.reshape(1, _FF)
    return _tc_combine(x, xs.reshape(_B * _KP, _N, _W), w0, wl, wh, bias2d)


# speedup vs baseline: 2.6789x; 2.6789x over previous
import functools

import jax
import jax.numpy as jnp
from jax import lax
from jax.experimental import pallas as pl
from jax.experimental.pallas import tpu as pltpu
from jax.experimental.pallas import tpu_sc as plsc

_B, _N, _F = 4, 10000, 128
_K, _FILT = 3, 4
_NNZ = 320000
_FF = _F * _FILT

_NC, _NS, _L = 2, 16, 16
_BPC = _B // _NC
_C = 128
_CHB = _NNZ // _C
_NCH = _CHB // _NS
_NXTRA = _CHB - _NCH * _NS
_NP = 10240
_RPT = _NP // _NS
_RC = 64
_NRC = _RPT // _RC
_PSTART = 9984
_NV = _F // _L


def _sc_body(xf, cols3, rows3, vals3, out,
             acc, gbuf, colb, rowb, valb, scat, rbuf,
             msc0, msc1, msr0, msr1, msv0, msv1, gs0, gs1, ss0, ss1):
    cid = lax.axis_index("c")
    sid = lax.axis_index("s")
    row0 = sid * _RPT
    msc, msr, msv = (msc0, msc1), (msr0, msr1), (msv0, msv1)
    gs, ss = (gs0, gs1), (ss0, ss1)

    def _zero_rbuf():
        zv = jnp.zeros((_L,), jnp.float32)

        def _zrow(i, carry):
            for q in range(_NV):
                rbuf[i, pl.ds(q * _L, _L)] = zv
            return carry

        lax.fori_loop(0, _RC, _zrow, 0)

    def _init_at(k, b, start, nr):
        if k == 1:
            pltpu.sync_copy(rbuf.at[pl.ds(0, nr)], acc.at[pl.ds(start, nr)])
        else:
            if k == 2:
                pltpu.sync_copy(xf.at[pl.ds(b * _N + start, nr)],
                                rbuf.at[pl.ds(0, nr)])
            else:
                pltpu.sync_copy(
                    out.at[pl.ds((b * _K + (k - 3)) * _N + start, nr)],
                    rbuf.at[pl.ds(0, nr)])

            def _neg(ii, carry):
                for q in range(_NV):
                    sl = pl.ds(q * _L, _L)
                    rbuf[ii, sl] = -rbuf[ii, sl]
                return carry

            lax.fori_loop(0, nr, _neg, 0)
            pltpu.sync_copy(rbuf.at[pl.ds(0, nr)], acc.at[pl.ds(start, nr)])

    def _wb_at(k, b, start, nr):
        pltpu.sync_copy(acc.at[pl.ds(start, nr)], rbuf.at[pl.ds(0, nr)])
        pltpu.sync_copy(
            rbuf.at[pl.ds(0, nr)],
            out.at[pl.ds((b * _K + (k - 1)) * _N + start, nr)])

    def _meta_start(cb, p):
        pltpu.async_copy(cols3.at[cb], colb.at[p], msc[p])
        pltpu.async_copy(rows3.at[cb], rowb.at[p], msr[p])
        pltpu.async_copy(vals3.at[cb], valb.at[p], msv[p])

    def _meta_wait(cb, p):
        pltpu.make_async_copy(cols3.at[cb], colb.at[p], msc[p]).wait()
        pltpu.make_async_copy(rows3.at[cb], rowb.at[p], msr[p]).wait()
        pltpu.make_async_copy(vals3.at[cb], valb.at[p], msv[p]).wait()

    def _adjust(p, offv):
        for q in range(_NV):
            sl = pl.ds(q * _L, _L)
            colb[p, 0, sl] = colb[p, 0, sl] + offv

    def _gather_start(tab, p):
        pltpu.async_copy(tab.at[colb.at[p, 0]], gbuf.at[p], gs[p])

    def _gather_wait(tab, p):
        pltpu.make_async_copy(tab.at[colb.at[p, 0]], gbuf.at[p], gs[p]).wait()

    def _rowcopy(p):
        for q in range(_NV):
            sl = pl.ds(q * _L, _L)
            scat[p, 0, sl] = rowb[p, 0, sl]

    def _scale(p, mul2):
        def _grp(g, carry):
            ev = valb[p, 0, pl.ds(g * _L, _L)]
            if mul2:
                ev = ev * 2.0
            for l in range(_L):
                vv = jnp.full((_L,), ev[l], dtype=jnp.float32)
                e = g * _L + l
                for q in range(_NV):
                    sl = pl.ds(q * _L, _L)
                    gbuf[p, e, sl] = gbuf[p, e, sl] * vv
            return carry

        lax.fori_loop(0, _C // _L, _grp, 0)

    def _scatter_start(p):
        pltpu.async_copy(gbuf.at[p], acc.at[scat.at[p, 0]], ss[p], add=True)

    def _scatter_wait(p):
        pltpu.make_async_copy(gbuf.at[p], acc.at[scat.at[p, 0]], ss[p]).wait()

    def _edge_phase(b, k, tab, tab_off, mul2):
        cb0 = b * _CHB + sid * _NCH
        offv = jnp.full((_L,), tab_off, dtype=jnp.int32)

        _meta_start(cb0, 0)
        _meta_start(cb0 + 1, 1)
        _meta_wait(cb0, 0)
        _adjust(0, offv)
        _gather_start(tab, 0)
        _meta_wait(cb0 + 1, 1)
        _adjust(1, offv)
        _gather_start(tab, 1)
        _gather_wait(tab, 0)
        _rowcopy(0)
        _scale(0, mul2)
        _scatter_start(0)
        _meta_start(cb0 + 2, 0)

        def _pair(i, carry):
            j0 = 2 * i
            for p in range(2):
                j = j0 + p
                o = 1 - p
                _meta_wait(cb0 + j, p)
                _scatter_wait(p)
                _adjust(p, offv)
                _gather_start(tab, p)
                _gather_wait(tab, o)
                _rowcopy(o)
                _scale(o, mul2)
                _scatter_start(o)
                _meta_start(cb0 + j + 1, o)
            return carry

        lax.fori_loop(1, _NCH // 2, _pair, 0)

        _meta_wait(cb0 + _NCH, 0)
        _scatter_wait(0)
        _gather_wait(tab, 1)
        _rowcopy(1)
        _scale(1, mul2)
        _scatter_start(1)
        _scatter_wait(1)

        @pl.when(sid < _NXTRA)
        def _():
            cbx = b * _CHB + _NCH * _NS + sid
            pltpu.sync_copy(cols3.at[cbx], colb.at[0])
            pltpu.sync_copy(rows3.at[cbx], rowb.at[0])
            pltpu.sync_copy(vals3.at[cbx], valb.at[0])
            _adjust(0, offv)
            pltpu.async_copy(tab.at[colb.at[0, 0]], gbuf.at[0], gs[0]).wait()
            _scale(0, mul2)
            pltpu.sync_copy(gbuf.at[0], acc.at[rowb.at[0, 0]], add=True)

    def _wbinit_at(k, b, start, nr):
        pltpu.sync_copy(acc.at[pl.ds(start, nr)], rbuf.at[pl.ds(0, nr)])
        pltpu.sync_copy(
            rbuf.at[pl.ds(0, nr)],
            out.at[pl.ds((b * _K + (k - 1)) * _N + start, nr)])
        if k == _K:
            return
        if k == 1:
            pltpu.sync_copy(xf.at[pl.ds(b * _N + start, nr)],
                            rbuf.at[pl.ds(0, nr)])
        else:
            pltpu.sync_copy(
                out.at[pl.ds((b * _K + (k - 2)) * _N + start, nr)],
                rbuf.at[pl.ds(0, nr)])

        def _neg(ii, carry):
            for q in range(_NV):
                sl = pl.ds(q * _L, _L)
                rbuf[ii, sl] = -rbuf[ii, sl]
            return carry

        lax.fori_loop(0, nr, _neg, 0)
        pltpu.sync_copy(rbuf.at[pl.ds(0, nr)], acc.at[pl.ds(start, nr)])

    def _graph(i, carry):
        b = cid * _BPC + i
        is_last_tile = row0 + _RPT > _N

        _zero_rbuf()
        for r in range(_NRC):
            start = row0 + r * _RC

            @pl.when(start + _RC <= _N)
            def _():
                pltpu.sync_copy(rbuf.at[pl.ds(0, _RC)],
                                acc.at[pl.ds(start, _RC)])

        @pl.when(is_last_tile)
        def _():
            pltpu.sync_copy(rbuf.at[pl.ds(0, _L)],
                            acc.at[pl.ds(_PSTART, _L)])

        for k in range(1, _K + 1):
            plsc.subcore_barrier()

            tab = xf if k == 1 else out
            tab_off = b * _N if k == 1 else (b * _K + (k - 2)) * _N
            _edge_phase(b, k, tab, tab_off, k >= 2)
            plsc.subcore_barrier()

            for r in range(_NRC):
                start = row0 + r * _RC

                @pl.when(start + _RC <= _N)
                def _():
                    _wbinit_at(k, b, start, _RC)

            @pl.when(is_last_tile)
            def _():
                _wbinit_at(k, b, _PSTART, _L)

        return carry

    lax.fori_loop(0, _BPC, _graph, 0)


_sc_cheb = functools.partial(
    pl.kernel,
    out_type=jax.ShapeDtypeStruct((_B * _K * _N, _F), jnp.float32),
    mesh=plsc.VectorSubcoreMesh(
        core_axis_name="c", subcore_axis_name="s",
        num_cores=_NC, num_subcores=_NS),
    scratch_types=[
        pltpu.VMEM_SHARED((_NP, _F), jnp.float32),
        pltpu.VMEM((2, _C, _F), jnp.float32),
        pltpu.VMEM((2, 1, _C), jnp.int32),
        pltpu.VMEM((2, 1, _C), jnp.int32),
        pltpu.VMEM((2, 1, _C), jnp.float32),
        pltpu.VMEM((2, 1, _C), jnp.int32),
        pltpu.VMEM((_RC, _F), jnp.float32),
        pltpu.SemaphoreType.DMA,
        pltpu.SemaphoreType.DMA,
        pltpu.SemaphoreType.DMA,
        pltpu.SemaphoreType.DMA,
        pltpu.SemaphoreType.DMA,
        pltpu.SemaphoreType.DMA,
        pltpu.SemaphoreType.DMA,
        pltpu.SemaphoreType.DMA,
        pltpu.SemaphoreType.DMA,
        pltpu.SemaphoreType.DMA,
    ],
)(_sc_body)


_BN = 400


def _combine_body(x_ref, xs_ref, w_ref, b_ref, o_ref):
    a = jnp.dot(x_ref[0], w_ref[0], preferred_element_type=jnp.float32)
    for k in range(1, _K + 1):
        a = a + jnp.dot(xs_ref[0, k - 1], w_ref[k],
                        preferred_element_type=jnp.float32)
    o_ref[0] = jnp.maximum(a + b_ref[:, :], 0.0)


def _tc_combine(x, xs, wbig, bias2d):
    return pl.pallas_call(
        _combine_body,
        grid=(_B, _N // _BN),
        in_specs=[
            pl.BlockSpec((1, _BN, _F), lambda b, n: (b, n, 0)),
            pl.BlockSpec((1, _K, _BN, _F), lambda b, n: (b, 0, n, 0)),
            pl.BlockSpec((_K + 1, _F, _FF), lambda b, n: (0, 0, 0)),
            pl.BlockSpec((1, _FF), lambda b, n: (0, 0)),
        ],
        out_specs=pl.BlockSpec((1, _BN, _FF), lambda b, n: (b, n, 0)),
        out_shape=jax.ShapeDtypeStruct((_B, _N, _FF), jnp.float32),
    )(x, xs, wbig, bias2d)


def kernel(x, lap_rows, lap_cols, lap_vals, weight, bias):
    xf = x.reshape(_B * _N, _F)
    cols3 = lap_cols.reshape(_B * _CHB, 1, _C).astype(jnp.int32)
    rows3 = lap_rows.reshape(_B * _CHB, 1, _C).astype(jnp.int32)
    vals3 = lap_vals.reshape(_B * _CHB, 1, _C)
    xs = _sc_cheb(xf, cols3, rows3, vals3)

    eye = jnp.eye(_F, dtype=jnp.float32)
    wbig = jnp.stack(
        [(eye[:, :, None] * weight[k][None, None, :]).reshape(_F, _FF)
         for k in range(_K + 1)])
    bias2d = bias.reshape(1, _FF)
    return _tc_combine(x, xs.reshape(_B, _K, _N, _F), wbig, bias2d)


# speedup vs baseline: 2.7108x; 1.0119x over previous
import functools

import jax
import jax.numpy as jnp
from jax import lax
from jax.experimental import pallas as pl
from jax.experimental.pallas import tpu as pltpu
from jax.experimental.pallas import tpu_sc as plsc

_B, _N, _F = 4, 10000, 128
_K, _FILT = 3, 4
_NNZ = 320000
_FF = _F * _FILT

_NC, _NS, _L = 2, 16, 16
_BPC = _B // _NC
_C = 128
_CHB = _NNZ // _C
_NCH = _CHB // _NS
_NXTRA = _CHB - _NCH * _NS
_NP = 10240
_RPT = _NP // _NS
_RC = 64
_NRC = _RPT // _RC
_PSTART = 9984
_NV = _F // _L


def _sc_body(xf, cols3, rows3, vals3, out,
             acc, gbuf, colb, rowb, valb, scat, rbuf,
             msc0, msc1, msr0, msr1, msv0, msv1, gs0, gs1, ss0, ss1):
    cid = lax.axis_index("c")
    sid = lax.axis_index("s")
    row0 = sid * _RPT
    msc, msr, msv = (msc0, msc1), (msr0, msr1), (msv0, msv1)
    gs, ss = (gs0, gs1), (ss0, ss1)

    def _zero_rbuf():
        zv = jnp.zeros((_L,), jnp.float32)

        def _zrow(i, carry):
            for q in range(_NV):
                rbuf[i, pl.ds(q * _L, _L)] = zv
            return carry

        lax.fori_loop(0, _RC, _zrow, 0)

    def _init_at(k, b, start, nr):
        if k == 1:
            pltpu.sync_copy(rbuf.at[pl.ds(0, nr)], acc.at[pl.ds(start, nr)])
        else:
            if k == 2:
                pltpu.sync_copy(xf.at[pl.ds(b * _N + start, nr)],
                                rbuf.at[pl.ds(0, nr)])
            else:
                pltpu.sync_copy(
                    out.at[pl.ds((b * _K + (k - 3)) * _N + start, nr)],
                    rbuf.at[pl.ds(0, nr)])

            def _neg(ii, carry):
                for q in range(_NV):
                    sl = pl.ds(q * _L, _L)
                    rbuf[ii, sl] = -rbuf[ii, sl]
                return carry

            lax.fori_loop(0, nr, _neg, 0)
            pltpu.sync_copy(rbuf.at[pl.ds(0, nr)], acc.at[pl.ds(start, nr)])

    def _wb_at(k, b, start, nr):
        pltpu.sync_copy(acc.at[pl.ds(start, nr)], rbuf.at[pl.ds(0, nr)])
        pltpu.sync_copy(
            rbuf.at[pl.ds(0, nr)],
            out.at[pl.ds((b * _K + (k - 1)) * _N + start, nr)])

    def _meta_start(cb, p):
        pltpu.async_copy(cols3.at[cb], colb.at[p], msc[p])
        pltpu.async_copy(rows3.at[cb], rowb.at[p], msr[p])
        pltpu.async_copy(vals3.at[cb], valb.at[p], msv[p])

    def _meta_wait(cb, p):
        pltpu.make_async_copy(cols3.at[cb], colb.at[p], msc[p]).wait()
        pltpu.make_async_copy(rows3.at[cb], rowb.at[p], msr[p]).wait()
        pltpu.make_async_copy(vals3.at[cb], valb.at[p], msv[p]).wait()

    def _adjust(p, offv):
        for q in range(_NV):
            sl = pl.ds(q * _L, _L)
            colb[p, 0, sl] = colb[p, 0, sl] + offv

    def _gather_start(tab, p):
        pltpu.async_copy(tab.at[colb.at[p, 0]], gbuf.at[p], gs[p])

    def _gather_wait(tab, p):
        pltpu.make_async_copy(tab.at[colb.at[p, 0]], gbuf.at[p], gs[p]).wait()

    def _rowcopy(p):
        for q in range(_NV):
            sl = pl.ds(q * _L, _L)
            scat[p, 0, sl] = rowb[p, 0, sl]

    def _scale(p, mul2):
        def _grp(g, carry):
            ev = valb[p, 0, pl.ds(g * _L, _L)]
            if mul2:
                ev = ev * 2.0
            for l in range(_L):
                vv = jnp.full((_L,), ev[l], dtype=jnp.float32)
                e = g * _L + l
                for q in range(_NV):
                    sl = pl.ds(q * _L, _L)
                    gbuf[p, e, sl] = gbuf[p, e, sl] * vv
            return carry

        lax.fori_loop(0, _C // _L, _grp, 0)

    def _scatter_start(p):
        pltpu.async_copy(gbuf.at[p], acc.at[scat.at[p, 0]], ss[p], add=True)

    def _scatter_wait(p):
        pltpu.make_async_copy(gbuf.at[p], acc.at[scat.at[p, 0]], ss[p]).wait()

    def _edge_phase(b, k, tab, tab_off, mul2):
        cb0 = b * _CHB + sid * _NCH
        offv = jnp.full((_L,), tab_off, dtype=jnp.int32)

        _meta_start(cb0, 0)
        _meta_start(cb0 + 1, 1)
        _meta_wait(cb0, 0)
        _adjust(0, offv)
        _gather_start(tab, 0)
        _meta_wait(cb0 + 1, 1)
        _adjust(1, offv)
        _gather_start(tab, 1)
        _gather_wait(tab, 0)
        _rowcopy(0)
        _scale(0, mul2)
        _scatter_start(0)
        _meta_start(cb0 + 2, 0)

        def _pair(i, carry):
            j0 = 2 * i
            for p in range(2):
                j = j0 + p
                o = 1 - p
                _meta_wait(cb0 + j, p)
                _scatter_wait(p)
                _adjust(p, offv)
                _gather_start(tab, p)
                _gather_wait(tab, o)
                _rowcopy(o)
                _scale(o, mul2)
                _scatter_start(o)
                _meta_start(cb0 + j + 1, o)
            return carry

        lax.fori_loop(1, _NCH // 2, _pair, 0)

        _meta_wait(cb0 + _NCH, 0)
        _scatter_wait(0)
        _gather_wait(tab, 1)
        _rowcopy(1)
        _scale(1, mul2)
        _scatter_start(1)
        _scatter_wait(1)

        @pl.when(sid < _NXTRA)
        def _():
            cbx = b * _CHB + _NCH * _NS + sid
            pltpu.sync_copy(cols3.at[cbx], colb.at[0])
            pltpu.sync_copy(rows3.at[cbx], rowb.at[0])
            pltpu.sync_copy(vals3.at[cbx], valb.at[0])
            _adjust(0, offv)
            pltpu.async_copy(tab.at[colb.at[0, 0]], gbuf.at[0], gs[0]).wait()
            _scale(0, mul2)
            pltpu.sync_copy(gbuf.at[0], acc.at[rowb.at[0, 0]], add=True)

    def _graph(i, carry):
        b = cid * _BPC + i
        is_last_tile = row0 + _RPT > _N
        for k in range(1, _K + 1):
            if k == 1:
                _zero_rbuf()
            for r in range(_NRC):
                start = row0 + r * _RC

                @pl.when(start + _RC <= _N)
                def _():
                    _init_at(k, b, start, _RC)

            @pl.when(is_last_tile)
            def _():
                _init_at(k, b, _PSTART, _L)

            plsc.subcore_barrier()

            tab = xf if k == 1 else out
            tab_off = b * _N if k == 1 else (b * _K + (k - 2)) * _N
            _edge_phase(b, k, tab, tab_off, k >= 2)
            plsc.subcore_barrier()

            for r in range(_NRC):
                start = row0 + r * _RC

                @pl.when(start + _RC <= _N)
                def _():
                    _wb_at(k, b, start, _RC)

            @pl.when(is_last_tile)
            def _():
                _wb_at(k, b, _PSTART, _L)

        return carry

    lax.fori_loop(0, _BPC, _graph, 0)


_sc_cheb = functools.partial(
    pl.kernel,
    out_type=jax.ShapeDtypeStruct((_B * _K * _N, _F), jnp.float32),
    mesh=plsc.VectorSubcoreMesh(
        core_axis_name="c", subcore_axis_name="s",
        num_cores=_NC, num_subcores=_NS),
    scratch_types=[
        pltpu.VMEM_SHARED((_NP, _F), jnp.float32),
        pltpu.VMEM((2, _C, _F), jnp.float32),
        pltpu.VMEM((2, 1, _C), jnp.int32),
        pltpu.VMEM((2, 1, _C), jnp.int32),
        pltpu.VMEM((2, 1, _C), jnp.float32),
        pltpu.VMEM((2, 1, _C), jnp.int32),
        pltpu.VMEM((_RC, _F), jnp.float32),
        pltpu.SemaphoreType.DMA,
        pltpu.SemaphoreType.DMA,
        pltpu.SemaphoreType.DMA,
        pltpu.SemaphoreType.DMA,
        pltpu.SemaphoreType.DMA,
        pltpu.SemaphoreType.DMA,
        pltpu.SemaphoreType.DMA,
        pltpu.SemaphoreType.DMA,
        pltpu.SemaphoreType.DMA,
        pltpu.SemaphoreType.DMA,
    ],
)(_sc_body)


_BN = 400


def _combine_body(x_ref, xs_ref, w_ref, b_ref, o_ref):
    a = jnp.dot(x_ref[0], w_ref[0], preferred_element_type=jnp.float32)
    for k in range(1, _K + 1):
        a = a + jnp.dot(xs_ref[0, k - 1], w_ref[k],
                        preferred_element_type=jnp.float32)
    o_ref[0] = jnp.maximum(a + b_ref[:, :], 0.0)


def _tc_combine(x, xs, wbig, bias2d):
    return pl.pallas_call(
        _combine_body,
        grid=(_B, _N // _BN),
        in_specs=[
            pl.BlockSpec((1, _BN, _F), lambda b, n: (b, n, 0)),
            pl.BlockSpec((1, _K, _BN, _F), lambda b, n: (b, 0, n, 0)),
            pl.BlockSpec((_K + 1, _F, _FF), lambda b, n: (0, 0, 0)),
            pl.BlockSpec((1, _FF), lambda b, n: (0, 0)),
        ],
        out_specs=pl.BlockSpec((1, _BN, _FF), lambda b, n: (b, n, 0)),
        out_shape=jax.ShapeDtypeStruct((_B, _N, _FF), jnp.float32),
    )(x, xs, wbig, bias2d)


def kernel(x, lap_rows, lap_cols, lap_vals, weight, bias):
    xf = x.reshape(_B * _N, _F)
    cols3 = lap_cols.reshape(_B * _CHB, 1, _C).astype(jnp.int32)
    rows3 = lap_rows.reshape(_B * _CHB, 1, _C).astype(jnp.int32)
    vals3 = lap_vals.reshape(_B * _CHB, 1, _C)
    xs = _sc_cheb(xf, cols3, rows3, vals3)

    eye = jnp.eye(_F, dtype=jnp.float32)
    wbig = jnp.stack(
        [(eye[:, :, None] * weight[k][None, None, :]).reshape(_F, _FF)
         for k in range(_K + 1)])
    bias2d = bias.reshape(1, _FF)
    return _tc_combine(x, xs.reshape(_B, _K, _N, _F), wbig, bias2d)
